# baseline (speedup 1.0000x reference)
import functools

import jax
import jax.numpy as jnp
from jax import lax
from jax.experimental import pallas as pl
from jax.experimental.pallas import tpu as pltpu
from jax.experimental.pallas import tpu_sc as plsc

NC = 2
NS = 16
BATCH = 128
CB = 16
LANES = 16


def _matmul_body(x_ref, w_ref, o_ref):
    dh = o_ref.shape[2]
    x = x_ref[...]
    o_ref[0] = jnp.dot(x, w_ref[:, :dh], preferred_element_type=jnp.float32)
    o_ref[1] = jnp.dot(x, w_ref[:, dh:], preferred_element_type=jnp.float32)


def _epilogue_body(acc_ref, deg_ref, b_ref, fc_ref, bf_ref, o_ref):
    n = o_ref.shape[0]
    s = jnp.concatenate([acc_ref[0, :n, :], acc_ref[1, :n, :]], axis=1)
    d = deg_ref[0, :n, 0:1] + deg_ref[1, :n, 0:1] + 1.0
    val = s / (d + 1e-16)
    val = jnp.where(jnp.isnan(val), 0.0, val)
    val = val + b_ref[...]
    g = jax.nn.sigmoid(
        jnp.sum(val * fc_ref[...], axis=1, keepdims=True) + bf_ref[...])
    o_ref[...] = (jnp.where(val < 0.0, 0.0, val)
                  + g * jnp.where(val > 0.0, 0.0, val))


def _make_sc_scatter(n_nodes, dh, nb, nr):
    rows_per_tile = nr // NS
    h_rows_per_tile = n_nodes // NS
    n_chunks = nb // CB
    np2 = n_chunks // 2
    mesh = plsc.VectorSubcoreMesh(core_axis_name="c", subcore_axis_name="s")

    @functools.partial(
        pl.kernel,
        out_type=(
            jax.ShapeDtypeStruct((NC, nr, dh), jnp.float32),
            jax.ShapeDtypeStruct((NC, nr, LANES), jnp.float32),
        ),
        mesh=mesh,
        compiler_params=pltpu.CompilerParams(use_tc_tiling_on_sc=False),
        scratch_types=(
            pltpu.VMEM((2, CB, BATCH), jnp.int32),
            pltpu.VMEM((2, CB, BATCH), jnp.int32),
            pltpu.VMEM((2, BATCH, dh), jnp.float32),
            pltpu.VMEM((BATCH, LANES), jnp.float32),
            pltpu.VMEM((128, dh), jnp.float32),
            pltpu.VMEM((128, LANES), jnp.float32),
            pltpu.VMEM_SHARED((nr, dh), jnp.float32),
            pltpu.VMEM_SHARED((nr, dh), jnp.float32),
            pltpu.VMEM_SHARED((nr, LANES), jnp.float32),
            pltpu.SemaphoreType.DMA,
            pltpu.SemaphoreType.DMA,
            pltpu.SemaphoreType.DMA,
            pltpu.SemaphoreType.DMA,
        ),
    )
    def sc_scatter(h_hbm, row_hbm, col_hbm, acc_out, deg_out,
                   row_v, col_v, rows_v, ones_v, zrow_v, zdeg_v,
                   h_sh, acc_sh, deg_sh, isem, gsem, ssem, dsem):
        cid = lax.axis_index("c")
        sid = lax.axis_index("s")

        hbase = sid * h_rows_per_tile
        pltpu.async_copy(h_hbm.at[cid, pl.ds(hbase, h_rows_per_tile)],
                         h_sh.at[pl.ds(hbase, h_rows_per_tile)], gsem)
        pltpu.async_copy(row_hbm.at[sid, pl.ds(0, CB)], row_v.at[0], isem)
        pltpu.async_copy(col_hbm.at[sid, pl.ds(0, CB)], col_v.at[0], isem)

        zeros16 = jnp.zeros((LANES,), jnp.float32)
        ones16 = jnp.ones((LANES,), jnp.float32)

        def init_zrow(j, carry):
            for k in range(dh // LANES):
                zrow_v[j, pl.ds(k * LANES, LANES)] = zeros16
            return carry
        lax.fori_loop(0, 128, init_zrow, 0)

        def init_small(j, carry):
            ones_v[j, pl.ds(0, LANES)] = ones16
            zdeg_v[j, pl.ds(0, LANES)] = zeros16
            return carry
        lax.fori_loop(0, 128, init_small, 0)

        @pl.when(sid == 0)
        def _zero_h_tail():
            off = n_nodes
            left = nr - n_nodes
            while left > 0:
                step = min(128, left)
                pltpu.sync_copy(zrow_v.at[pl.ds(0, step)],
                                h_sh.at[pl.ds(off, step)])
                off += step
                left -= step

        base = sid * rows_per_tile
        for t in range(rows_per_tile // 128):
            pltpu.sync_copy(zdeg_v, deg_sh.at[pl.ds(base + t * 128, 128)])

        pltpu.make_async_copy(
            h_hbm.at[cid, pl.ds(hbase, h_rows_per_tile)],
            h_sh.at[pl.ds(hbase, h_rows_per_tile)], gsem).wait()
        plsc.subcore_barrier()

        for t in range(rows_per_tile // 128):
            pltpu.sync_copy(h_sh.at[pl.ds(base + t * 128, 128)], zrow_v)
            pltpu.sync_copy(zrow_v, acc_sh.at[pl.ds(base + t * 128, 128)])
        plsc.subcore_barrier()

        dummy = jnp.full((LANES,), n_nodes, jnp.int32)

        def remap(pb):
            def body(i, carry):
                b = i // 8
                k = (i % 8) * LANES
                r = row_v[pb, b, pl.ds(k, LANES)]
                c = col_v[pb, b, pl.ds(k, LANES)]
                row_v[pb, b, pl.ds(k, LANES)] = jnp.where(r == c, dummy, r)
                return carry
            lax.fori_loop(0, CB * 8, body, 0)

        def process_chunk(p, pb, with_deg, mid=None):
            sdesc = [None, None]
            ddesc = [None]
            prev_g = None

            def g_issue(b):
                return pltpu.async_copy(
                    h_sh.at[col_v.at[pb, b]], rows_v.at[b % 2], gsem)

            def s_issue(b):
                return pltpu.async_copy(
                    rows_v.at[b % 2], acc_sh.at[row_v.at[pb, b]], ssem,
                    add=True)

            def wait_slot(i):
                if sdesc[i] is not None:
                    sdesc[i].wait()
                    sdesc[i] = None

            def scatter(b):
                sdesc[b % 2] = s_issue(b)
                if with_deg:
                    if ddesc[0] is not None:
                        ddesc[0].wait()
                    ddesc[0] = pltpu.async_copy(
                        ones_v, deg_sh.at[row_v.at[pb, b]], dsem, add=True)

            for b in range(CB):
                wait_slot(b % 2)
                g = g_issue(b)
                if prev_g is not None:
                    prev_g.wait()
                    scatter(b - 1)
                prev_g = g
                if b == 3 and mid is not None:
                    mid()
            prev_g.wait()
            wait_slot((CB - 1) % 2)
            scatter(CB - 1)
            wait_slot(0)
            wait_slot(1)
            if with_deg and ddesc[0] is not None:
                ddesc[0].wait()

        def run(deg_parity):
            def idx_drain(pb):
                pltpu.make_async_copy(row_hbm.at[sid, pl.ds(0, CB)],
                                      row_v.at[pb], isem).wait()
                pltpu.make_async_copy(col_hbm.at[sid, pl.ds(0, CB)],
                                      col_v.at[pb], isem).wait()

            def idx_issue(c, pb):
                pltpu.async_copy(row_hbm.at[sid, pl.ds(c * CB, CB)],
                                 row_v.at[pb], isem)
                pltpu.async_copy(col_hbm.at[sid, pl.ds(c * CB, CB)],
                                 col_v.at[pb], isem)

            def pair(p, carry):
                c0 = 2 * p

                def mid0():
                    idx_drain(1)
                    remap(1)

                process_chunk(p, 0, deg_parity == 0, mid0)

                @pl.when(p + 1 < np2)
                def _():
                    idx_issue(c0 + 2, 0)

                def mid1():
                    @pl.when(p + 1 < np2)
                    def _():
                        idx_drain(0)
                        remap(0)

                process_chunk(p, 1, deg_parity == 1, mid1)

                @pl.when(p + 1 < np2)
                def _():
                    idx_issue(c0 + 3, 1)
                return carry

            idx_drain(0)
            idx_issue(1, 1)
            remap(0)
            lax.fori_loop(0, np2, pair, 0)

        @pl.when(cid == 0)
        def _core0():
            run(0)

        @pl.when(cid == 1)
        def _core1():
            run(1)

        plsc.subcore_barrier()

        pltpu.sync_copy(acc_sh.at[pl.ds(base, rows_per_tile)],
                        acc_out.at[cid, pl.ds(base, rows_per_tile)])
        pltpu.sync_copy(deg_sh.at[pl.ds(base, rows_per_tile)],
                        deg_out.at[cid, pl.ds(base, rows_per_tile)])

    return sc_scatter


def kernel(x, edge_index, W, bias, att_p, fc, bf):
    n, d_in = x.shape
    d = W.shape[1]
    dh = d // 2
    e = edge_index.shape[1]

    h = pl.pallas_call(
        _matmul_body,
        out_shape=jax.ShapeDtypeStruct((NC, n, dh), jnp.float32),
    )(x, W)

    e_per_t = -(-e // NS)
    nb = 2 * CB * (-(-e_per_t // (2 * CB * BATCH)))
    e_pad = NS * nb * BATCH
    row = edge_index[0].astype(jnp.int32)
    col = edge_index[1].astype(jnp.int32)
    pad = e_pad - e
    row_p = jnp.concatenate(
        [row, jnp.full((pad,), n, jnp.int32)]).reshape(NS, nb, BATCH)
    col_p = jnp.concatenate(
        [col, jnp.zeros((pad,), jnp.int32)]).reshape(NS, nb, BATCH)

    nr = -(-(n + 1) // (128 * NS)) * (128 * NS)

    acc, deg = _make_sc_scatter(n, dh, nb, nr)(h, row_p, col_p)

    out = pl.pallas_call(
        _epilogue_body,
        out_shape=jax.ShapeDtypeStruct((n, d), jnp.float32),
    )(acc, deg, bias.reshape(1, d), fc.reshape(1, d), bf.reshape(1, 1))
    return out
